# baseline (speedup 1.0000x reference)
import jax
import jax.numpy as jnp
from jax import lax
from jax.experimental import pallas as pl
from jax.experimental.pallas import tpu as pltpu
from jax.experimental.pallas import tpu_sc as plsc

_MAX_H = 64
_MAX_W = 64
_S = _MAX_H * _MAX_W
_B = 4096
_NC, _NS, _L = 2, 16, 16
_NW = _NC * _NS
_RPT = _B // _NW
_ICH = 8
_OCH = 4
_NCH = _RPT // _OCH
_NCI = _RPT // _ICH
_NBI = 2
_NBO = 3
_GROUPS = _S // _L


def _sc_body(x_hbm, off_hbm, pe_hbm, out_hbm,
             pe_raw, pe_s, off_v, buf_in, buf_out, sem_in, sem_out):
    wid = lax.axis_index("s") * _NC + lax.axis_index("c")
    base = wid * _RPT

    def in_cp(c):
        b = lax.rem(c, _NBI)
        return pltpu.make_async_copy(
            x_hbm.at[pl.ds(base + c * _ICH, _ICH)], buf_in.at[b],
            sem_in.at[b])

    def out_cp(c):
        b = lax.rem(c, _NBO)
        return pltpu.make_async_copy(
            buf_out.at[b], out_hbm.at[pl.ds(base + c * _OCH, _OCH)],
            sem_out.at[b])

    for b in range(_NBI):
        in_cp(b).start()

    pltpu.sync_copy(pe_hbm, pe_raw)
    pltpu.sync_copy(off_hbm, off_v)
    lanes = lax.iota(jnp.int32, _L)
    off = off_v[...]

    def gather_body(g, carry):
        s = pl.ds(g * _L, _L)
        iv = jnp.clip(lanes + (g * _L + off), 0, _S - 1)
        pe_s[s] = plsc.load_gather(pe_raw, [iv]) * jnp.float32(0.1)
        return carry

    lax.fori_loop(0, _GROUPS, gather_body, 0)

    def chunk_body(c, carry):
        cin = lax.div(c, 2)
        half = lax.rem(c, 2)
        bi = lax.rem(cin, _NBI)
        bo = lax.rem(c, _NBO)

        @pl.when(half == 0)
        def _():
            in_cp(cin).wait()

        @pl.when(c >= _NBO)
        def _():
            out_cp(c - _NBO).wait()

        rbase = half * _OCH

        @plsc.parallel_loop(0, _GROUPS, unroll=2)
        def add_body(g):
            s = pl.ds(g * _L, _L)
            pe_vec = pe_s[s]
            for r in range(_OCH):
                buf_out[bo, r, s] = buf_in[bi, rbase + r, s] + pe_vec
        out_cp(c).start()

        @pl.when((half == 1) & (cin + _NBI < _NCI))
        def _():
            in_cp(cin + _NBI).start()

        return carry

    lax.fori_loop(0, _NCH, chunk_body, 0)

    def drain_body(c, carry):
        out_cp(c).wait()
        return carry

    lax.fori_loop(_NCH - _NBO, _NCH, drain_body, 0)


def kernel(x_flat, height, width, pos_embed):
    offset = (jnp.asarray(height, jnp.int32) - _MAX_H) + (
        jnp.asarray(width, jnp.int32) - _MAX_W
    )
    off_vec = jnp.full((_L,), offset, dtype=jnp.int32)
    run = pl.kernel(
        _sc_body,
        out_type=jax.ShapeDtypeStruct((_B, _S), jnp.float32),
        mesh=plsc.VectorSubcoreMesh(core_axis_name="c", subcore_axis_name="s"),
        compiler_params=pltpu.CompilerParams(needs_layout_passes=False),
        scratch_types=[
            pltpu.VMEM((_S,), jnp.float32),
            pltpu.VMEM((_S,), jnp.float32),
            pltpu.VMEM((_L,), jnp.int32),
            pltpu.VMEM((_NBI, _ICH, _S), jnp.float32),
            pltpu.VMEM((_NBO, _OCH, _S), jnp.float32),
            pltpu.SemaphoreType.DMA((_NBI,)),
            pltpu.SemaphoreType.DMA((_NBO,)),
        ],
    )
    return run(x_flat, off_vec, pos_embed)


# speedup vs baseline: 1.0055x; 1.0055x over previous
import jax
import jax.numpy as jnp
from jax import lax
from jax.experimental import pallas as pl
from jax.experimental.pallas import tpu as pltpu
from jax.experimental.pallas import tpu_sc as plsc

_MAX_H = 64
_MAX_W = 64
_S = _MAX_H * _MAX_W
_B = 4096
_NC, _NS, _L = 2, 16, 16
_NW = _NC * _NS
_RPT = _B // _NW
_CHUNK = 4
_NCH = _RPT // _CHUNK
_NBUF = 3
_GROUPS = _S // _L


def _sc_body(x_hbm, off_hbm, pe_hbm, out_hbm,
             pe_raw, pe_s, off_v, buf_in, buf_out, sem_in, sem_out):
    wid = lax.axis_index("s") * _NC + lax.axis_index("c")
    base = wid * _RPT

    def in_cp(c, b):
        return pltpu.make_async_copy(
            x_hbm.at[pl.ds(base + c * _CHUNK, _CHUNK)], buf_in.at[b],
            sem_in.at[b])

    def out_cp(c, b):
        return pltpu.make_async_copy(
            buf_out.at[b], out_hbm.at[pl.ds(base + c * _CHUNK, _CHUNK)],
            sem_out.at[b])

    for b in range(_NBUF):
        in_cp(b, b).start()

    pltpu.sync_copy(pe_hbm, pe_raw)
    pltpu.sync_copy(off_hbm, off_v)
    lanes = lax.iota(jnp.int32, _L)
    off = off_v[...]

    def gather_body(g, carry):
        s = pl.ds(g * _L, _L)
        iv = jnp.clip(lanes + (g * _L + off), 0, _S - 1)
        pe_s[s] = plsc.load_gather(pe_raw, [iv]) * jnp.float32(0.1)
        return carry

    lax.fori_loop(0, _GROUPS, gather_body, 0)

    def chunk_body(c, carry):
        b = lax.rem(c, _NBUF)
        in_cp(c, b).wait()

        @pl.when(c >= _NBUF)
        def _():
            out_cp(c - _NBUF, b).wait()

        @plsc.parallel_loop(0, _GROUPS, unroll=8)
        def add_body(g):
            s = pl.ds(g * _L, _L)
            pe_vec = pe_s[s]
            for r in range(_CHUNK):
                buf_out[b, r, s] = buf_in[b, r, s] + pe_vec
        out_cp(c, b).start()

        @pl.when(c + _NBUF < _NCH)
        def _():
            in_cp(c + _NBUF, b).start()

        return carry

    lax.fori_loop(0, _NCH, chunk_body, 0)

    def drain_body(c, carry):
        out_cp(c, lax.rem(c, _NBUF)).wait()
        return carry

    lax.fori_loop(_NCH - _NBUF, _NCH, drain_body, 0)


def kernel(x_flat, height, width, pos_embed):
    offset = (jnp.asarray(height, jnp.int32) - _MAX_H) + (
        jnp.asarray(width, jnp.int32) - _MAX_W
    )
    off_vec = jnp.full((_L,), offset, dtype=jnp.int32)
    run = pl.kernel(
        _sc_body,
        out_type=jax.ShapeDtypeStruct((_B, _S), jnp.float32),
        mesh=plsc.VectorSubcoreMesh(core_axis_name="c", subcore_axis_name="s"),
        compiler_params=pltpu.CompilerParams(needs_layout_passes=False),
        scratch_types=[
            pltpu.VMEM((_S,), jnp.float32),
            pltpu.VMEM((_S,), jnp.float32),
            pltpu.VMEM((_L,), jnp.int32),
            pltpu.VMEM((_NBUF, _CHUNK, _S), jnp.float32),
            pltpu.VMEM((_NBUF, _CHUNK, _S), jnp.float32),
            pltpu.SemaphoreType.DMA((_NBUF,)),
            pltpu.SemaphoreType.DMA((_NBUF,)),
        ],
    )
    return run(x_flat, off_vec, pos_embed)


# speedup vs baseline: 1.0140x; 1.0084x over previous
import jax
import jax.numpy as jnp
from jax import lax
from jax.experimental import pallas as pl
from jax.experimental.pallas import tpu as pltpu
from jax.experimental.pallas import tpu_sc as plsc

_MAX_H = 64
_MAX_W = 64
_S = _MAX_H * _MAX_W
_B = 4096
_NC, _NS, _L = 2, 16, 16
_NW = _NC * _NS
_RPT = _B // _NW
_CHUNK = 4
_NCH = _RPT // _CHUNK
_NBUF = 3
_GROUPS = _S // _L


def _sc_body(x_hbm, off_hbm, pe_hbm, out_hbm,
             pe_raw, pe_s, off_v, buf_in, buf_out, sem_in, sem_out):
    wid = lax.axis_index("s") * _NC + lax.axis_index("c")
    base = wid * _RPT

    def in_cp(c, b):
        return pltpu.make_async_copy(
            x_hbm.at[pl.ds(base + c * _CHUNK, _CHUNK)], buf_in.at[b],
            sem_in.at[b])

    def out_cp(c, b):
        return pltpu.make_async_copy(
            buf_out.at[b], out_hbm.at[pl.ds(base + c * _CHUNK, _CHUNK)],
            sem_out.at[b])

    for b in range(_NBUF):
        in_cp(b, b).start()

    pltpu.sync_copy(pe_hbm, pe_raw)
    pltpu.sync_copy(off_hbm, off_v)
    lanes = lax.iota(jnp.int32, _L)
    off = off_v[...]

    def gather_body(g, carry):
        s = pl.ds(g * _L, _L)
        iv = jnp.clip(lanes + (g * _L + off), 0, _S - 1)
        pe_s[s] = plsc.load_gather(pe_raw, [iv]) * jnp.float32(0.1)
        return carry

    lax.fori_loop(0, _GROUPS, gather_body, 0)

    def chunk_body(c, carry):
        b = lax.rem(c, _NBUF)
        in_cp(c, b).wait()

        @pl.when(c >= _NBUF)
        def _():
            out_cp(c - _NBUF, b).wait()

        @plsc.parallel_loop(0, _GROUPS, unroll=2)
        def add_body(g):
            s = pl.ds(g * _L, _L)
            pe_vec = pe_s[s]
            for r in range(_CHUNK):
                buf_out[b, r, s] = buf_in[b, r, s] + pe_vec
        out_cp(c, b).start()

        @pl.when(c + _NBUF < _NCH)
        def _():
            in_cp(c + _NBUF, b).start()

        return carry

    lax.fori_loop(0, _NCH, chunk_body, 0)

    def drain_body(c, carry):
        out_cp(c, lax.rem(c, _NBUF)).wait()
        return carry

    lax.fori_loop(_NCH - _NBUF, _NCH, drain_body, 0)


def kernel(x_flat, height, width, pos_embed):
    offset = (jnp.asarray(height, jnp.int32) - _MAX_H) + (
        jnp.asarray(width, jnp.int32) - _MAX_W
    )
    off_vec = jnp.full((_L,), offset, dtype=jnp.int32)
    run = pl.kernel(
        _sc_body,
        out_type=jax.ShapeDtypeStruct((_B, _S), jnp.float32),
        mesh=plsc.VectorSubcoreMesh(core_axis_name="c", subcore_axis_name="s"),
        compiler_params=pltpu.CompilerParams(needs_layout_passes=False),
        scratch_types=[
            pltpu.VMEM((_S,), jnp.float32),
            pltpu.VMEM((_S,), jnp.float32),
            pltpu.VMEM((_L,), jnp.int32),
            pltpu.VMEM((_NBUF, _CHUNK, _S), jnp.float32),
            pltpu.VMEM((_NBUF, _CHUNK, _S), jnp.float32),
            pltpu.SemaphoreType.DMA((_NBUF,)),
            pltpu.SemaphoreType.DMA((_NBUF,)),
        ],
    )
    return run(x_flat, off_vec, pos_embed)
